# Initial kernel scaffold; baseline (speedup 1.0000x reference)
import jax
import jax.numpy as jnp
from jax.experimental import pallas as pl


def kernel(pred, lbl):
    raise NotImplementedError("write your pallas kernel here")


# speedup vs baseline: 40.6722x; 40.6722x over previous
import functools

import jax
import jax.numpy as jnp
from jax import lax
from jax.experimental import pallas as pl
from jax.experimental.pallas import tpu as pltpu
from jax.experimental.pallas import tpu_sc as plsc

NBINS = 2048
NCLS = 19
NC, NS, L = 2, 16, 16
NW = NC * NS
HSIZE = NCLS * 2 * NBINS
UNROLL = 8


def _sc_hist(pred3, lbl2):
    B, C, HW = pred3.shape
    CH = (B * HW) // NW
    per_b = HW // CH
    mesh = plsc.VectorSubcoreMesh(core_axis_name="c", subcore_axis_name="s")

    @functools.partial(
        pl.kernel,
        mesh=mesh,
        out_type=jax.ShapeDtypeStruct((NW, HSIZE), jnp.float32),
        scratch_types=[
            pltpu.VMEM((HSIZE,), jnp.float32),
            pltpu.VMEM((CH,), jnp.int32),
            pltpu.VMEM((CH,), jnp.float32),
        ],
        compiler_params=pltpu.CompilerParams(needs_layout_passes=False),
    )
    def hist_kernel(pred_hbm, lbl_hbm, out_hbm, histv, lblv, predv):
        wid = lax.axis_index("s") * NC + lax.axis_index("c")
        b = wid // per_b
        h0 = (wid % per_b) * CH

        zero16 = jnp.zeros((L,), jnp.float32)
        one16 = jnp.ones((L,), jnp.float32)

        def zbody(i, _):
            base = i * (L * UNROLL)
            for u in range(UNROLL):
                histv[pl.ds(base + u * L, L)] = zero16
            return 0

        lax.fori_loop(0, HSIZE // (L * UNROLL), zbody, 0)

        pltpu.sync_copy(lbl_hbm.at[b, pl.ds(h0, CH)], lblv)

        def cbody(c, _):
            pltpu.sync_copy(pred_hbm.at[b, c, pl.ds(h0, CH)], predv)
            cbase = c * (2 * NBINS)

            def ibody(i, _):
                base = i * (L * UNROLL)
                for u in range(UNROLL):
                    off = base + u * L
                    p = predv[pl.ds(off, L)]
                    lb = lblv[pl.ds(off, L)]
                    fg = lb == c
                    e = jnp.where(fg, 1.0 - p, p)
                    bi = jnp.minimum((e * float(NBINS)).astype(jnp.int32),
                                     NBINS - 1)
                    idx = cbase + jnp.where(fg, NBINS, 0) + bi
                    plsc.addupdate_scatter(histv, [idx], one16)
                return 0

            lax.fori_loop(0, CH // (L * UNROLL), ibody, 0)
            return 0

        lax.fori_loop(0, C, cbody, 0)
        pltpu.sync_copy(histv, out_hbm.at[wid])

    return hist_kernel(pred3, lbl2)


def _tc_finish(h0, h1):
    def body(h0_ref, h1_ref, out_ref):
        n0 = jnp.sum(h0_ref[...], axis=0)
        n1 = jnp.sum(h1_ref[...], axis=0)
        row = lax.broadcasted_iota(jnp.int32, (NBINS, NBINS), 0)
        col = lax.broadcasted_iota(jnp.int32, (NBINS, NBINS), 1)
        tri = (row <= col).astype(jnp.float32)
        dn = (((1,), (0,)), ((), ()))
        cum0 = lax.dot_general(n0, tri, dn, precision=lax.Precision.HIGHEST,
                               preferred_element_type=jnp.float32)
        cum1 = lax.dot_general(n1, tri, dn, precision=lax.Precision.HIGHEST,
                               preferred_element_type=jnp.float32)
        tot0 = jnp.sum(n0, axis=1, keepdims=True)
        tot1 = jnp.sum(n1, axis=1, keepdims=True)
        gts = tot1
        above0 = tot0 - cum0
        above1 = tot1 - cum1
        i_start = above0 + above1
        u_start = gts + above0
        j_start = i_start / jnp.maximum(u_start, 1.0)
        u_mid = u_start + n0
        j_mid = (i_start + n0) / jnp.maximum(u_mid, 1.0)
        center = (lax.broadcasted_iota(jnp.int32, (NCLS, NBINS), 1)
                  .astype(jnp.float32) + 0.5) * (1.0 / NBINS)
        contrib = center * ((j_mid - j_start) + n1 / jnp.maximum(u_mid, 1.0))
        loss_c = jnp.sum(contrib, axis=1, keepdims=True)
        pres = (gts > 0.0).astype(jnp.float32)
        npres = jnp.maximum(jnp.sum(pres, axis=0, keepdims=True), 1.0)
        num = jnp.sum(loss_c * pres, axis=0, keepdims=True)
        out_ref[...] = num / npres

    return pl.pallas_call(
        body,
        out_shape=jax.ShapeDtypeStruct((1, 1), jnp.float32),
    )(h0, h1)


def kernel(pred, lbl):
    B, C, H, W = pred.shape
    pred3 = pred.reshape(B, C, H * W)
    lbl2 = lbl.reshape(B, H * W).astype(jnp.int32)
    hist = _sc_hist(pred3, lbl2).reshape(NW, NCLS, 2, NBINS)
    loss = _tc_finish(hist[:, :, 0], hist[:, :, 1])
    return loss.reshape(())


# speedup vs baseline: 122.2786x; 3.0064x over previous
import functools

import jax
import jax.numpy as jnp
from jax import lax
from jax.experimental import pallas as pl
from jax.experimental.pallas import tpu as pltpu
from jax.experimental.pallas import tpu_sc as plsc

NBINS = 2048
NCLS = 19
NC, NS, L = 2, 16, 16
NW = NC * NS
HSIZE = 2 * NCLS * NBINS
UNROLL = 8


def _sc_hist(pred3, lbl2):
    B, C, HW = pred3.shape
    CH = (B * HW) // NW
    per_b = HW // CH
    mesh = plsc.VectorSubcoreMesh(core_axis_name="c", subcore_axis_name="s")

    @functools.partial(
        pl.kernel,
        mesh=mesh,
        out_type=jax.ShapeDtypeStruct((NW, HSIZE), jnp.float32),
        scratch_types=[
            pltpu.VMEM((HSIZE,), jnp.float32),
            pltpu.VMEM((CH,), jnp.int32),
            pltpu.VMEM((CH,), jnp.float32),
        ],
        compiler_params=pltpu.CompilerParams(needs_layout_passes=False),
    )
    def hist_kernel(pred_hbm, lbl_hbm, out_hbm, histv, lblv, predv):
        wid = lax.axis_index("s") * NC + lax.axis_index("c")
        b = wid // per_b
        h0 = (wid % per_b) * CH

        zero16 = jnp.zeros((L,), jnp.float32)
        one16 = jnp.ones((L,), jnp.float32)

        @functools.partial(plsc.parallel_loop, 0, HSIZE // L, unroll=UNROLL)
        def _zero(i):
            histv[pl.ds(i * L, L)] = zero16

        pltpu.sync_copy(lbl_hbm.at[b, pl.ds(h0, CH)], lblv)

        def cbody(c, _):
            pltpu.sync_copy(pred_hbm.at[b, c, pl.ds(h0, CH)], predv)
            hbase = c * NBINS
            gbase = NCLS * NBINS + c * NBINS

            @functools.partial(plsc.parallel_loop, 0, CH // L,
                               unroll=UNROLL)
            def _accum(i):
                off = i * L
                p = predv[pl.ds(off, L)]
                lb = lblv[pl.ds(off, L)]
                bi = jnp.minimum((p * float(NBINS)).astype(jnp.int32),
                                 NBINS - 1)
                plsc.addupdate_scatter(histv, [hbase + bi], one16)
                plsc.addupdate_scatter(histv, [gbase + bi], one16,
                                       mask=lb == c)

            return 0

        lax.fori_loop(0, C, cbody, 0)
        pltpu.sync_copy(histv, out_hbm.at[wid])

    return hist_kernel(pred3, lbl2)


def _tc_finish(hh, gh, ghf):
    def body(h_ref, g_ref, gf_ref, out_ref):
        hsum = jnp.sum(h_ref[...], axis=0)
        gsum = jnp.sum(g_ref[...], axis=0)
        n0 = hsum - gsum
        n1 = jnp.sum(gf_ref[...], axis=0)
        row = lax.broadcasted_iota(jnp.int32, (NBINS, NBINS), 0)
        col = lax.broadcasted_iota(jnp.int32, (NBINS, NBINS), 1)
        tri = (row <= col).astype(jnp.float32)
        dn = (((1,), (0,)), ((), ()))
        both = jnp.concatenate([n0, n1], axis=0)
        cums = lax.dot_general(both, tri, dn, precision=lax.Precision.HIGHEST,
                               preferred_element_type=jnp.float32)
        cum0 = cums[:NCLS]
        cum1 = cums[NCLS:]
        tot0 = jnp.sum(n0, axis=1, keepdims=True)
        tot1 = jnp.sum(n1, axis=1, keepdims=True)
        gts = tot1
        above0 = tot0 - cum0
        above1 = tot1 - cum1
        i_start = above0 + above1
        u_start = gts + above0
        j_start = i_start / jnp.maximum(u_start, 1.0)
        u_mid = u_start + n0
        j_mid = (i_start + n0) / jnp.maximum(u_mid, 1.0)
        center = (lax.broadcasted_iota(jnp.int32, (NCLS, NBINS), 1)
                  .astype(jnp.float32) + 0.5) * (1.0 / NBINS)
        contrib = center * ((j_mid - j_start) + n1 / jnp.maximum(u_mid, 1.0))
        loss_c = jnp.sum(contrib, axis=1, keepdims=True)
        pres = (gts > 0.0).astype(jnp.float32)
        npres = jnp.maximum(jnp.sum(pres, axis=0, keepdims=True), 1.0)
        num = jnp.sum(loss_c * pres, axis=0, keepdims=True)
        out_ref[...] = num / npres

    return pl.pallas_call(
        body,
        out_shape=jax.ShapeDtypeStruct((1, 1), jnp.float32),
    )(hh, gh, ghf)


def kernel(pred, lbl):
    B, C, H, W = pred.shape
    pred3 = pred.reshape(B, C, H * W)
    lbl2 = lbl.reshape(B, H * W).astype(jnp.int32)
    hist = _sc_hist(pred3, lbl2).reshape(NW, 2, NCLS, NBINS)
    gh = hist[:, 1]
    loss = _tc_finish(hist[:, 0], gh, jnp.flip(gh, axis=-1))
    return loss.reshape(())


# speedup vs baseline: 158.8273x; 1.2989x over previous
import functools

import jax
import jax.numpy as jnp
from jax import lax
from jax.experimental import pallas as pl
from jax.experimental.pallas import tpu as pltpu
from jax.experimental.pallas import tpu_sc as plsc

NBINS = 2048
NCLS = 19
NC, NS, L = 2, 16, 16
NW = NC * NS
NROWS = 2 * NCLS
HSIZE = NROWS * NBINS
UNROLL = 8


def _sc_hist(pred3, lbl2):
    B, C, HW = pred3.shape
    CH = (B * HW) // NW
    per_b = HW // CH
    mesh = plsc.VectorSubcoreMesh(core_axis_name="c", subcore_axis_name="s")

    @functools.partial(
        pl.kernel,
        mesh=mesh,
        out_type=jax.ShapeDtypeStruct((NW, NROWS, NBINS), jnp.float32),
        scratch_types=[
            pltpu.VMEM((HSIZE,), jnp.float32),
            pltpu.VMEM((CH,), jnp.int32),
            pltpu.VMEM((CH,), jnp.float32),
            pltpu.SemaphoreType.DMA,
        ],
        compiler_params=pltpu.CompilerParams(needs_layout_passes=False),
    )
    def hist_kernel(pred_hbm, lbl_hbm, out_hbm, histv, lblv, predv, dsem):
        wid = lax.axis_index("s") * NC + lax.axis_index("c")
        b = wid // per_b
        h0 = (wid % per_b) * CH

        zero16 = jnp.zeros((L,), jnp.float32)
        one16 = jnp.ones((L,), jnp.float32)

        @functools.partial(plsc.parallel_loop, 0, HSIZE // L, unroll=UNROLL)
        def _zero(i):
            histv[pl.ds(i * L, L)] = zero16

        pltpu.sync_copy(lbl_hbm.at[b, pl.ds(h0, CH)], lblv)

        def cbody(c, _):
            pltpu.sync_copy(pred_hbm.at[b, c, pl.ds(h0, CH)], predv)
            hbase = c * NBINS
            gbase = (NCLS + c) * NBINS

            @functools.partial(plsc.parallel_loop, 0, CH // L,
                               unroll=UNROLL)
            def _accum(i):
                off = i * L
                p = predv[pl.ds(off, L)]
                lb = lblv[pl.ds(off, L)]
                bi = jnp.minimum((p * float(NBINS)).astype(jnp.int32),
                                 NBINS - 1)
                base = jnp.where(lb == c, gbase, hbase)
                plsc.addupdate_scatter(histv, [base + bi], one16)

            return 0

        lax.fori_loop(0, C, cbody, 0)

        copies = [
            pltpu.async_copy(histv.at[pl.ds(r * NBINS, NBINS)],
                             out_hbm.at[wid, r], dsem)
            for r in range(NROWS)
        ]
        for cp in copies:
            cp.wait()

    return hist_kernel(pred3, lbl2)


def _tc_finish(hist3):
    def body(h_ref, out_ref):
        s = jnp.sum(h_ref[...], axis=0)
        n0 = s[:NCLS]
        gsum = s[NCLS:]
        row = lax.broadcasted_iota(jnp.int32, (NBINS, NBINS), 0)
        col = lax.broadcasted_iota(jnp.int32, (NBINS, NBINS), 1)
        tri = (row <= col).astype(jnp.float32)
        trir = (row + col >= NBINS - 1).astype(jnp.float32)
        dn = (((1,), (0,)), ((), ()))
        cum0 = lax.dot_general(n0, tri, dn, precision=lax.Precision.HIGHEST,
                               preferred_element_type=jnp.float32)
        cum1 = lax.dot_general(gsum, trir, dn,
                               precision=lax.Precision.HIGHEST,
                               preferred_element_type=jnp.float32)
        n1 = cum1 - jnp.concatenate(
            [jnp.zeros((NCLS, 1), jnp.float32), cum1[:, :NBINS - 1]], axis=1)
        tot0 = jnp.sum(n0, axis=1, keepdims=True)
        tot1 = jnp.sum(gsum, axis=1, keepdims=True)
        gts = tot1
        above0 = tot0 - cum0
        above1 = tot1 - cum1
        i_start = above0 + above1
        u_start = gts + above0
        j_start = i_start / jnp.maximum(u_start, 1.0)
        u_mid = u_start + n0
        j_mid = (i_start + n0) / jnp.maximum(u_mid, 1.0)
        center = (lax.broadcasted_iota(jnp.int32, (NCLS, NBINS), 1)
                  .astype(jnp.float32) + 0.5) * (1.0 / NBINS)
        contrib = center * ((j_mid - j_start) + n1 / jnp.maximum(u_mid, 1.0))
        loss_c = jnp.sum(contrib, axis=1, keepdims=True)
        pres = (gts > 0.0).astype(jnp.float32)
        npres = jnp.maximum(jnp.sum(pres, axis=0, keepdims=True), 1.0)
        num = jnp.sum(loss_c * pres, axis=0, keepdims=True)
        out_ref[...] = num / npres

    return pl.pallas_call(
        body,
        out_shape=jax.ShapeDtypeStruct((1, 1), jnp.float32),
    )(hist3)


def kernel(pred, lbl):
    B, C, H, W = pred.shape
    pred3 = pred.reshape(B, C, H * W)
    lbl2 = lbl.reshape(B, H * W).astype(jnp.int32)
    loss = _tc_finish(_sc_hist(pred3, lbl2))
    return loss.reshape(())
